# baseline (speedup 1.0000x reference)
import jax
import jax.numpy as jnp
from jax.experimental import pallas as pl


def _slice_kernel(x_ref, o_ref):
    o_ref[...] = x_ref[:, :, 1:6, :]


def kernel(x):
    return pl.pallas_call(
        _slice_kernel,
        out_shape=jax.ShapeDtypeStruct((128, 1, 5, 5), x.dtype),
        grid=(1,),
        in_specs=[pl.BlockSpec((128, 1, 8, 5), lambda i: (0, 0, 0, 0))],
        out_specs=pl.BlockSpec((128, 1, 5, 5), lambda i: (0, 0, 0, 0)),
    )(x)


# speedup vs baseline: 13.4877x; 13.4877x over previous
import jax
import jax.numpy as jnp
from jax.experimental import pallas as pl


def _slice_kernel(xt_ref, o_ref):
    for i in range(5):
        o_ref[:, :, i, :] = xt_ref[:, :, :, 1 + i]


def kernel(x):
    xt = jnp.swapaxes(x, 2, 3)
    return pl.pallas_call(
        _slice_kernel,
        out_shape=jax.ShapeDtypeStruct((128, 1, 5, 5), x.dtype),
        grid=(1,),
        in_specs=[pl.BlockSpec((128, 1, 5, 128), lambda i: (0, 0, 0, 0))],
        out_specs=pl.BlockSpec((128, 1, 5, 5), lambda i: (0, 0, 0, 0)),
    )(xt)
